# baseline (speedup 1.0000x reference)
import jax
import jax.numpy as jnp
from jax import lax
from jax.experimental import pallas as pl
from jax.experimental.pallas import tpu as pltpu
from jax.experimental.pallas import tpu_sc as plsc

NC = 2
NS = 16
NW = NC * NS
LANES = 16

B = 16384
L = 50
E = 64
BW = B // NW
CB = 256
NBLK = CB // LANES
SUB = 128
LN_EPS = 1e-5


def _rsqrt(x):
    i = plsc.bitcast(x, jnp.int32)
    y = plsc.bitcast(jnp.int32(0x5F3759DF) - (i >> 1), jnp.float32)
    for _ in range(3):
        y = y * (1.5 - 0.5 * x * y * y)
    return y


def _body(idx_hbm, info_hbm, table_hbm, gamma_hbm, beta_hbm, out_hbm,
          idx_all, info_all, idx_buf, rows_v, a_v, b_v, gb_v, gbb_v, stg_v,
          sem):
    wid = lax.axis_index("s") * NC + lax.axis_index("c")
    nw_base = pl.multiple_of(wid * (BW * L), BW * L)

    pltpu.sync_copy(idx_hbm.at[pl.ds(nw_base, BW * L)], idx_all)
    pltpu.sync_copy(info_hbm.at[pl.ds(nw_base, BW * L)], info_all)
    pltpu.sync_copy(gamma_hbm, gb_v.at[0])
    pltpu.sync_copy(beta_hbm, gb_v.at[1])

    iota16 = lax.iota(jnp.int32, LANES)
    iota_l = iota16 * L

    def bcast_body(e, carry):
        ebc = jnp.full((LANES,), e, jnp.int32)
        gbb_v[e, pl.ds(0, LANES)] = plsc.load_gather(gb_v, [jnp.zeros((LANES,), jnp.int32), ebc])
        gbb_v[E + e, pl.ds(0, LANES)] = plsc.load_gather(gb_v, [jnp.ones((LANES,), jnp.int32), ebc])
        return carry

    lax.fori_loop(0, E, bcast_body, 0)

    def chunk_body(c, carry):
        l = c // 2
        sb = c % 2
        cbase = sb * (CB * L) + l

        def idxb_body(k, carry2):
            pvec = iota_l + (cbase + k * (LANES * L))
            idx_buf[pl.ds(k * LANES, LANES)] = plsc.load_gather(idx_all, [pvec])
            return carry2

        lax.fori_loop(0, NBLK, idxb_body, 0)

        cps = [pltpu.async_copy(table_hbm.at[idx_buf.at[pl.ds(k * SUB, SUB)]],
                                rows_v.at[pl.ds(k * SUB, SUB)], sem)
               for k in range(CB // SUB)]
        for cp in cps:
            cp.wait()

        def stats_body(k, carry2):
            row0 = k * LANES
            rows16 = row0 + iota16
            s = jnp.zeros((LANES,), jnp.float32)
            s2 = jnp.zeros((LANES,), jnp.float32)
            for j in range(E):
                colj = jnp.full((LANES,), j, jnp.int32)
                v = plsc.load_gather(rows_v, [rows16, colj])
                s = s + v
                s2 = s2 + v * v
            mean = s * (1.0 / E)
            var_t = s2 * (1.0 / E) - mean * mean
            pvec = iota_l + (cbase + k * (LANES * L))
            infov = plsc.load_gather(info_all, [pvec])
            vy = jnp.maximum(infov * infov * var_t + LN_EPS, 1e-30)
            a = infov * _rsqrt(vy)
            a_v[pl.ds(row0, LANES)] = a
            b_v[pl.ds(row0, LANES)] = -mean * a
            return carry2

        lax.fori_loop(0, NBLK, stats_body, 0)

        def apply_body(k, carry2):
            row0 = k * LANES
            rows16 = row0 + iota16
            av = a_v[pl.ds(row0, LANES)]
            bv = b_v[pl.ds(row0, LANES)]
            tc = k // 8
            boff = (k % 8) * LANES
            for e in range(E):
                cole = jnp.full((LANES,), e, jnp.int32)
                x = plsc.load_gather(rows_v, [rows16, cole])
                ge = gbb_v[e, pl.ds(0, LANES)]
                be = gbb_v[E + e, pl.ds(0, LANES)]
                y = (x * av + bv) * ge + be
                stg_v[e // 8, tc, e % 8, pl.ds(boff, LANES)] = y
            return carry2

        lax.fori_loop(0, NBLK, apply_body, 0)

        btile0 = wid * (BW // 128) + sb * (CB // 128)
        pltpu.sync_copy(stg_v, out_hbm.at[l, :, pl.ds(btile0, CB // 128)])
        return carry

    lax.fori_loop(0, 2 * L, chunk_body, 0)


def kernel(holder, info, table, ln_gamma, ln_beta):
    b, l = holder.shape
    v, e = table.shape
    n = b * l
    assert (b, l, e) == (B, L, E)
    idx = holder.reshape(n).astype(jnp.int32)
    infof = info.reshape(n)

    mesh = plsc.VectorSubcoreMesh(core_axis_name="c", subcore_axis_name="s",
                                  num_cores=NC, num_subcores=NS)
    run = pl.kernel(
        _body,
        out_type=jax.ShapeDtypeStruct((L, E // 8, B // 128, 8, 128), jnp.float32),
        mesh=mesh,
        scratch_types=[
            pltpu.VMEM((BW * L,), jnp.int32),
            pltpu.VMEM((BW * L,), jnp.float32),
            pltpu.VMEM((CB,), jnp.int32),
            pltpu.VMEM((CB, E), jnp.float32),
            pltpu.VMEM((CB,), jnp.float32),
            pltpu.VMEM((CB,), jnp.float32),
            pltpu.VMEM((2, E), jnp.float32),
            pltpu.VMEM((2 * E, LANES), jnp.float32),
            pltpu.VMEM((E // 8, CB // 128, 8, 128), jnp.float32),
            pltpu.SemaphoreType.DMA,
        ],
        compiler_params=pltpu.CompilerParams(needs_layout_passes=False,
                                             use_tc_tiling_on_sc=False),
    )
    out5 = run(idx, infof, table, ln_gamma, ln_beta)
    return out5.transpose(2, 4, 0, 1, 3).reshape(B, L, E)


# speedup vs baseline: 1.2241x; 1.2241x over previous
import jax
import jax.numpy as jnp
from jax import lax
from jax.experimental import pallas as pl
from jax.experimental.pallas import tpu as pltpu
from jax.experimental.pallas import tpu_sc as plsc

NC = 2
NS = 16
NW = NC * NS
LANES = 16

B = 16384
L = 50
E = 64
EV = E // LANES
BW = B // NW
CB = 256
NBLK = CB // LANES
SUB = 128
LN_EPS = 1e-5


def _rsqrt(x):
    i = plsc.bitcast(x, jnp.int32)
    y = plsc.bitcast(jnp.int32(0x5F3759DF) - (i >> 1), jnp.float32)
    for _ in range(3):
        y = y * (1.5 - 0.5 * x * y * y)
    return y


def _body(idx_hbm, info_hbm, table_hbm, gamma_hbm, beta_hbm, out_hbm,
          idx_all, info_all, idx_buf, rows_v, gb_v, stg_v, sums_v, ab_v, sem):
    wid = lax.axis_index("s") * NC + lax.axis_index("c")
    nw_base = pl.multiple_of(wid * (BW * L), BW * L)

    pltpu.sync_copy(idx_hbm.at[pl.ds(nw_base, BW * L)], idx_all)
    pltpu.sync_copy(info_hbm.at[pl.ds(nw_base, BW * L)], info_all)
    pltpu.sync_copy(gamma_hbm, gb_v.at[0])
    pltpu.sync_copy(beta_hbm, gb_v.at[1])
    gammas = [gb_v[0, pl.ds(p * LANES, LANES)] for p in range(EV)]
    betas = [gb_v[1, pl.ds(p * LANES, LANES)] for p in range(EV)]

    iota16 = lax.iota(jnp.int32, LANES)
    iota_l = iota16 * L

    svecs = [((16 * p + iota16) // 8) * 2048 + ((16 * p + iota16) % 8) * 128
             for p in range(EV)]

    def chunk_body(c, carry):
        l = c // 2
        sb = c % 2
        cbase = sb * (CB * L) + l

        def idxb_body(k, carry2):
            pvec = iota_l + (cbase + k * (LANES * L))
            idx_buf[pl.ds(k * LANES, LANES)] = plsc.load_gather(idx_all, [pvec])
            return carry2

        lax.fori_loop(0, NBLK, idxb_body, 0)

        cps = [pltpu.async_copy(table_hbm.at[idx_buf.at[pl.ds(k * SUB, SUB)]],
                                rows_v.at[pl.ds(k * SUB, SUB)], sem)
               for k in range(CB // SUB)]
        for cp in cps:
            cp.wait()

        def row_body(k, carry2):
            row0 = k * LANES
            rows16 = row0 + iota16
            info16 = plsc.load_gather(info_all,
                                      [iota_l + (cbase + k * (LANES * L))])
            off_k = (row0 // 128) * 1024 + (row0 % 128)
            ss = [jnp.zeros((LANES,), jnp.float32) for _ in range(4)]
            qq = [jnp.zeros((LANES,), jnp.float32) for _ in range(4)]
            for j in range(E):
                colj = jnp.full((LANES,), j, jnp.int32)
                v = plsc.load_gather(rows_v, [rows16, colj])
                ss[j % 4] = ss[j % 4] + v
                qq[j % 4] = qq[j % 4] + v * v
            svec = (ss[0] + ss[1]) + (ss[2] + ss[3])
            qvec = (qq[0] + qq[1]) + (qq[2] + qq[3])
            mean = svec * (1.0 / E)
            var_t = qvec * (1.0 / E) - mean * mean
            vy = jnp.maximum(info16 * info16 * var_t + LN_EPS, 1e-30)
            a_vec = info16 * _rsqrt(vy)
            ab_v[pl.ds(0, LANES)] = a_vec
            ab_v[pl.ds(LANES, LANES)] = -mean * a_vec
            for u in range(LANES):
                r = row0 + u
                ubc = jnp.full((LANES,), u, jnp.int32)
                abc = plsc.load_gather(ab_v, [ubc])
                bbc = plsc.load_gather(ab_v, [ubc + LANES])
                xs = [rows_v[r, pl.ds(p * LANES, LANES)] for p in range(EV)]
                for p in range(EV):
                    y = (xs[p] * abc + bbc) * gammas[p] + betas[p]
                    plsc.store_scatter(stg_v, [svecs[p] + (off_k + u)], y)
            return carry2

        lax.fori_loop(0, NBLK, row_body, 0)

        btile0 = wid * (BW // 128) + sb * (CB // 128)
        ocps = [pltpu.async_copy(
                    stg_v.at[pl.ds(tr * 2048, 2048)],
                    out_hbm.at[l, pl.ds(tr * (1024 * B // 128) + btile0 * 1024,
                                        2048)],
                    sem)
                for tr in range(E // 8)]
        for cp in ocps:
            cp.wait()
        return carry

    lax.fori_loop(0, 2 * L, chunk_body, 0)


def kernel(holder, info, table, ln_gamma, ln_beta):
    b, l = holder.shape
    v, e = table.shape
    n = b * l
    assert (b, l, e) == (B, L, E)
    idx = holder.reshape(n).astype(jnp.int32)
    infof = info.reshape(n)

    mesh = plsc.VectorSubcoreMesh(core_axis_name="c", subcore_axis_name="s",
                                  num_cores=NC, num_subcores=NS)
    run = pl.kernel(
        _body,
        out_type=jax.ShapeDtypeStruct((L, (E // 8) * (B // 128) * 8 * 128),
                                      jnp.float32),
        mesh=mesh,
        scratch_types=[
            pltpu.VMEM((BW * L,), jnp.int32),
            pltpu.VMEM((BW * L,), jnp.float32),
            pltpu.VMEM((CB,), jnp.int32),
            pltpu.VMEM((CB, E), jnp.float32),
            pltpu.VMEM((2, E), jnp.float32),
            pltpu.VMEM(((E // 8) * (CB // 128) * 8 * 128,), jnp.float32),
            pltpu.VMEM((2 * LANES, LANES), jnp.float32),
            pltpu.VMEM((2 * LANES,), jnp.float32),
            pltpu.SemaphoreType.DMA,
        ],
        compiler_params=pltpu.CompilerParams(needs_layout_passes=False,
                                             use_tc_tiling_on_sc=False),
    )
    out2 = run(idx, infof, table, ln_gamma, ln_beta)
    out5 = out2.reshape(L, E // 8, B // 128, 8, 128)
    return out5.transpose(2, 4, 0, 1, 3).reshape(B, L, E)
